# baseline (speedup 1.0000x reference)
import functools

import jax
import jax.numpy as jnp
from jax import lax
from jax.experimental import pallas as pl
from jax.experimental.pallas import tpu as pltpu
from jax.experimental.pallas import tpu_sc as plsc

B, T, F, VOCAB, D = 1024, 50, 26, 1000000, 16
ROW = 1 + D
NBT = B // 128
NTT = (T + 7) // 8
TPAD = NTT * 8
NCHUNK = T * NBT
NW = 32
CPW = -(-NCHUNK // NW)
IDS = F * 128
EVR = IDS // 16


PCV = 2000
NPC = VOCAB // PCV


@functools.cache
def _build_pack_sc():
  mesh = plsc.VectorSubcoreMesh(core_axis_name="c", subcore_axis_name="s")
  return functools.partial(
      pl.kernel,
      out_type=jax.ShapeDtypeStruct((VOCAB, D), jnp.float32),
      mesh=mesh,
      compiler_params=pltpu.CompilerParams(
          needs_layout_passes=False, use_tc_tiling_on_sc=False),
      scratch_types=[
          pltpu.VMEM((D, PCV), jnp.float32),
          pltpu.VMEM((PCV, D), jnp.float32),
      ],
  )(_pack_sc)


def _pack_sc(tabt_hbm, out_hbm, tin, tout):
  w = lax.axis_index("s") * 2 + lax.axis_index("c")
  dvec = lax.iota(jnp.int32, 16)

  def chunk_body(i, carry):
    cid = i * NW + w

    @pl.when(cid < NPC)
    def _():
      v0 = cid * PCV
      pltpu.sync_copy(tabt_hbm.at[:, pl.ds(v0, PCV)], tin)

      def vb_body(vb, vc):
        for dv in range(8):
          v = vb * 8 + dv
          tout[v, :] = plsc.load_gather(tin, [dvec, jnp.full((16,), 0, jnp.int32) + v])
        return vc

      lax.fori_loop(0, PCV // 8, vb_body, 0)
      pltpu.sync_copy(tout, out_hbm.at[pl.ds(v0, PCV)])

    return carry

  lax.fori_loop(0, -(-NPC // NW), chunk_body, 0)


def _pack_table(table):
  return _build_pack_sc()(table.T)


@functools.cache
def _build_embed_sc():
  mesh = plsc.VectorSubcoreMesh(core_axis_name="c", subcore_axis_name="s")
  return functools.partial(
      pl.kernel,
      out_type=jax.ShapeDtypeStruct((ROW, F, TPAD, B), jnp.float32),
      mesh=mesh,
      compiler_params=pltpu.CompilerParams(
          needs_layout_passes=False, use_tc_tiling_on_sc=False),
      scratch_types=[
          pltpu.VMEM((F, 256), jnp.float32),
          pltpu.VMEM((IDS, 16), jnp.float32),
          pltpu.VMEM((ROW, F, 128), jnp.float32),
          pltpu.VMEM((F, 128), jnp.int32),
          pltpu.SemaphoreType.DMA,
      ],
  )(_embed_sc)


def _embed_sc(x_hbm, tab_hbm, out_hbm, xbuf, rows, obuf, idxb, sem):
  w = lax.axis_index("s") * 2 + lax.axis_index("c")

  def chunk_body(i, carry):
    cid = i * NW + w

    @pl.when(cid < NCHUNK)
    def _():
      t = cid // NBT
      bt = cid % NBT

      pltpu.sync_copy(x_hbm.at[t, :, bt, :], xbuf)

      for k in range(EVR):
        f, j = k // 8, k % 8
        v = xbuf[f, pl.ds(128 + j * 16, 16)]
        idxb[f, pl.ds(j * 16, 16)] = v.astype(jnp.int32)

      handles = [
          pltpu.async_copy(
              tab_hbm.at[idxb.at[f]],
              rows.at[pl.ds(f * 128, 128)],
              sem,
          )
          for f in range(F)
      ]

      for k in range(EVR):
        f, j = k // 8, k % 8
        obuf[0, f, pl.ds(j * 16, 16)] = xbuf[f, pl.ds(j * 16, 16)]

      for h in handles:
        h.wait()

      rowv0 = lax.iota(jnp.int32, 16)

      def mbody(m, mc):
        rowv = rowv0 + m * 16
        f = m // 8
        col = (m % 8) * 16
        for r in range(1, ROW):
          v = plsc.load_gather(rows, [rowv, jnp.full((16,), r - 1, jnp.int32)])
          obuf[r, f, pl.ds(col, 16)] = v
        return mc

      lax.fori_loop(0, EVR, mbody, 0)

      pltpu.sync_copy(obuf, out_hbm.at[:, :, t, pl.ds(bt * 128, 128)])

    return carry

  lax.fori_loop(0, CPW, chunk_body, 0)


def kernel(X, table):
  x4 = (X.transpose(1, 3, 2, 0)
          .reshape(T, F, 2, NBT, 128)
          .transpose(0, 1, 3, 2, 4)
          .reshape(T, F, NBT, 256))
  out = _build_embed_sc()(x4, _pack_table(table))
  return out.transpose(3, 2, 0, 1)[:, :T]


# speedup vs baseline: 2.1631x; 2.1631x over previous
import functools

import jax
import jax.numpy as jnp
from jax import lax
from jax.experimental import pallas as pl
from jax.experimental.pallas import tpu as pltpu
from jax.experimental.pallas import tpu_sc as plsc

B, T, F, VOCAB, D = 1024, 50, 26, 1000000, 16
ROW = 1 + D
NBT = B // 128
TPAD = 56
NCHUNK = T * NBT
NW = 32
NSLOT = 14
IDS = F * 128
EVR = IDS // 16


@functools.cache
def _build_embed_sc():
  mesh = plsc.VectorSubcoreMesh(core_axis_name="c", subcore_axis_name="s")
  return functools.partial(
      pl.kernel,
      out_type=jax.ShapeDtypeStruct((ROW, F, TPAD, B), jnp.float32),
      mesh=mesh,
      compiler_params=pltpu.CompilerParams(
          needs_layout_passes=False, use_tc_tiling_on_sc=False),
      scratch_types=[
          pltpu.VMEM((F, 256), jnp.float32),
          pltpu.VMEM((IDS, 16), jnp.float32),
          pltpu.VMEM((ROW, F, 128), jnp.float32),
          pltpu.VMEM((F, 128), jnp.int32),
          pltpu.SemaphoreType.DMA,
          pltpu.SemaphoreType.DMA,
      ],
  )(_embed_sc)


def _embed_sc(x_hbm, tab_hbm, out_hbm, xbuf, rows, obuf, idxb, sem, semo):
  w = lax.axis_index("s") * 2 + lax.axis_index("c")
  rowv0 = lax.iota(jnp.int32, 16)

  def prep(cid):
    @pl.when(cid < NCHUNK)
    def _():
      t = cid // NBT
      bt = cid % NBT
      pltpu.sync_copy(x_hbm.at[t, :, bt, :], xbuf)
      for k in range(EVR):
        f, j = k // 8, k % 8
        v = xbuf[f, pl.ds(128 + j * 16, 16)]
        idxb[f, pl.ds(j * 16, 16)] = v.astype(jnp.int32)
      for f in range(F):
        pltpu.async_copy(
            tab_hbm.at[idxb.at[f]], rows.at[pl.ds(f * 128, 128)], sem)

  def body(s, carry):
    cid = s * NW + w

    @pl.when(cid < NCHUNK)
    def _consume():
      t = cid // NBT
      bt = cid % NBT
      for f in range(F):
        pltpu.make_async_copy(
            tab_hbm.at[idxb.at[f]], rows.at[pl.ds(f * 128, 128)], sem).wait()

      for k in range(EVR):
        f, j = k // 8, k % 8
        obuf[0, f, pl.ds(j * 16, 16)] = xbuf[f, pl.ds(j * 16, 16)]

      def mbody(mm, mc):
        for u in range(2):
          m = mm * 2 + u
          rowv = rowv0 + m * 16
          f = m // 8
          col = (m % 8) * 16
          for r in range(1, ROW):
            v = plsc.load_gather(
                rows, [rowv, jnp.full((16,), r - 1, jnp.int32)])
            obuf[r, f, pl.ds(col, 16)] = v
        return mc

      lax.fori_loop(0, EVR // 2, mbody, 0)
      pltpu.async_copy(
          obuf, out_hbm.at[:, :, t, pl.ds(bt * 128, 128)], semo)

    prep(cid + NW)

    @pl.when(cid < NCHUNK)
    def _drain():
      t = cid // NBT
      bt = cid % NBT
      pltpu.make_async_copy(
          obuf, out_hbm.at[:, :, t, pl.ds(bt * 128, 128)], semo).wait()

    return carry

  prep(w)
  lax.fori_loop(0, NSLOT, body, 0)


def kernel(X, table):
  x4 = (X.transpose(1, 3, 2, 0)
          .reshape(T, F, 2, NBT, 128)
          .transpose(0, 1, 3, 2, 4)
          .reshape(T, F, NBT, 256))
  out = _build_embed_sc()(x4, table)
  return out.transpose(3, 2, 0, 1)[:, :T]
